# baseline (speedup 1.0000x reference)
import jax
import jax.numpy as jnp
from jax.experimental import pallas as pl

N = 100000
D = 10
DOUT = 10


def _final_linear_body(agg_ref, w_ref, b_ref, out_ref):
    agg = agg_ref[...]
    out_ref[...] = jnp.dot(agg, w_ref[...].T, preferred_element_type=jnp.float32) + b_ref[...]


def _final_linear(agg, W2, b2):
    BN = 10000
    grid = (N // BN,)
    return pl.pallas_call(
        _final_linear_body,
        grid=grid,
        in_specs=[
            pl.BlockSpec((BN, D), lambda i: (i, 0)),
            pl.BlockSpec((DOUT, D), lambda i: (0, 0)),
            pl.BlockSpec((DOUT,), lambda i: (0,)),
        ],
        out_specs=pl.BlockSpec((BN, DOUT), lambda i: (i, 0)),
        out_shape=jax.ShapeDtypeStruct((N, DOUT), jnp.float32),
    )(agg, W2, b2)


def kernel(x, edge_index, mask, W1, b1, W2, b2):
    src = edge_index[0]
    dst = edge_index[1]
    x_i = jnp.take(x, dst, axis=0)
    x_j = jnp.take(x, src, axis=0)
    msg = jnp.concatenate([x_i, x_j - x_i], axis=-1) @ W1.T + b1
    agg = jax.ops.segment_max(msg, dst, num_segments=N)
    agg = jnp.where(jnp.isneginf(agg), 0.0, agg)
    return _final_linear(agg, W2, b2)


# speedup vs baseline: 5.4294x; 5.4294x over previous
import jax
import jax.numpy as jnp
from jax import lax
from jax.experimental import pallas as pl
from jax.experimental.pallas import tpu as pltpu
from jax.experimental.pallas import tpu_sc as plsc

N = 100000
E = 3200000
D = 10
DOUT = 10
DP = 16

NTILES = 32
NPAD = 100096
ROWS = NPAD // NTILES
SHIFT = 131072
CAP = E

ECHUNK = 6400
NECH = E // ECHUNK

SCHUNK = 4000
NSCH = (E // NTILES) // SCHUNK

GCHUNK = 1024

_SC_PARAMS = pltpu.CompilerParams(use_tc_tiling_on_sc=False)


def _prep_body(dst_ref, src_ref, rec_ref, sidx_ref, deg_ref, base_ref):
    c = pl.program_id(0)

    @pl.when(c == 0)
    def _():
        base_ref[...] = jnp.zeros((NTILES, 128), jnp.int32)

    d = dst_ref[0]
    s = src_ref[0]
    df = d.astype(jnp.float32)
    o = jnp.floor(df * (1.0 / ROWS)).astype(jnp.int32)
    o = o - (d < o * ROWS).astype(jnp.int32)
    o = o + (d >= (o + 1) * ROWS).astype(jnp.int32)
    loc = d - o * ROWS
    rec_ref[0] = loc * SHIFT + s

    tids = lax.broadcasted_iota(jnp.int32, (NTILES, ECHUNK), 0)
    oh = (o == tids).astype(jnp.int32)
    cs = oh
    sh = 1
    while sh < ECHUNK:
        z = jnp.zeros((NTILES, sh), jnp.int32)
        cs = cs + jnp.concatenate([z, cs[:, :-sh]], axis=1)
        sh *= 2
    rank = jnp.sum((cs - 1) * oh, axis=0, keepdims=True)
    base = base_ref[:, :1]
    off = jnp.sum(base * oh, axis=0, keepdims=True)
    sidx_ref[0] = o * CAP + off + rank
    tot = cs[:, -1:]
    newbase = base + tot
    base_ref[...] = jnp.broadcast_to(newbase, (NTILES, 128))
    deg_ref[...] = jnp.broadcast_to(newbase, (NTILES, 128))


def _prep(dst3, src3):
    return pl.pallas_call(
        _prep_body,
        grid=(NECH,),
        in_specs=[
            pl.BlockSpec((1, 1, ECHUNK), lambda c: (c, 0, 0)),
            pl.BlockSpec((1, 1, ECHUNK), lambda c: (c, 0, 0)),
        ],
        out_specs=[
            pl.BlockSpec((1, 1, ECHUNK), lambda c: (c, 0, 0)),
            pl.BlockSpec((1, 1, ECHUNK), lambda c: (c, 0, 0)),
            pl.BlockSpec((NTILES, 128), lambda c: (0, 0)),
        ],
        out_shape=[
            jax.ShapeDtypeStruct((NECH, 1, ECHUNK), jnp.int32),
            jax.ShapeDtypeStruct((NECH, 1, ECHUNK), jnp.int32),
            jax.ShapeDtypeStruct((NTILES, 128), jnp.int32),
        ],
        scratch_shapes=[pltpu.VMEM((NTILES, 128), jnp.int32)],
    )(dst3, src3)


def _scatter_body(rec_hbm, sidx_hbm, binned_hbm, recbuf, sidxbuf, sem_i, sem_o):
    wid = lax.axis_index("s") * 2 + lax.axis_index("c")
    ebase = wid * (E // NTILES)

    def chunk(ch, _):
        off = ebase + ch * SCHUNK
        pltpu.make_async_copy(rec_hbm.at[pl.ds(off, SCHUNK)], recbuf, sem_i).start()
        pltpu.make_async_copy(sidx_hbm.at[pl.ds(off, SCHUNK)], sidxbuf, sem_i).start()
        pltpu.make_async_copy(rec_hbm.at[pl.ds(off, SCHUNK)], recbuf, sem_i).wait()
        pltpu.make_async_copy(sidx_hbm.at[pl.ds(off, SCHUNK)], sidxbuf, sem_i).wait()
        pltpu.make_async_copy(recbuf, binned_hbm.at[sidxbuf], sem_o).start()
        pltpu.make_async_copy(recbuf, binned_hbm.at[sidxbuf], sem_o).wait()
        return 0

    lax.fori_loop(0, NSCH, chunk, 0)


def _scatter(rec, sidx):
    mesh = plsc.VectorSubcoreMesh(core_axis_name="c", subcore_axis_name="s")
    return pl.kernel(
        _scatter_body,
        out_type=jax.ShapeDtypeStruct((NTILES * CAP,), jnp.int32),
        mesh=mesh,
        compiler_params=_SC_PARAMS,
        scratch_types=[
            pltpu.VMEM((SCHUNK,), jnp.int32),
            pltpu.VMEM((SCHUNK,), jnp.int32),
            pltpu.SemaphoreType.DMA,
            pltpu.SemaphoreType.DMA,
        ],
    )(rec, sidx)


def _segmax_body(binned_hbm, deg_hbm, qp_hbm, s_hbm,
                 recbuf, idxbuf, qbuf, acc, degbuf, sem):
    wid = lax.axis_index("s") * 2 + lax.axis_index("c")
    base = wid * ROWS
    lid = lax.iota(jnp.int32, 16)

    neg = jnp.full((DP,), -jnp.inf, dtype=jnp.float32)

    def init_acc(r, _):
        acc[r] = neg
        return 0

    lax.fori_loop(0, ROWS, init_acc, 0)

    pltpu.make_async_copy(deg_hbm, degbuf, sem).start()
    pltpu.make_async_copy(deg_hbm, degbuf, sem).wait()
    k = degbuf[pl.ds(wid, 16)][0]

    nch = (k + (GCHUNK - 1)) // GCHUNK

    def chunk(ch, _):
        off = wid * CAP + ch * GCHUNK
        pltpu.make_async_copy(
            binned_hbm.at[pl.ds(off, GCHUNK)], recbuf.at[pl.ds(0, GCHUNK)], sem
        ).start()
        pltpu.make_async_copy(
            binned_hbm.at[pl.ds(off, GCHUNK)], recbuf.at[pl.ds(0, GCHUNK)], sem
        ).wait()
        kc = jnp.minimum(k - ch * GCHUNK, GCHUNK)

        def mkidx(v, _):
            r16 = recbuf[pl.ds(v * 16, 16)]
            i16 = r16 & (SHIFT - 1)
            valid = (v * 16 + lid) < kc
            idxbuf[pl.ds(v * 16, 16)] = jnp.where(valid, i16, lid)
            return 0

        lax.fori_loop(0, GCHUNK // 16, mkidx, 0)

        pltpu.make_async_copy(qp_hbm.at[idxbuf], qbuf, sem).start()
        pltpu.make_async_copy(qp_hbm.at[idxbuf], qbuf, sem).wait()

        def rmw(j, _):
            r = recbuf[pl.ds(j, 16)][0]
            i = lax.shift_right_logical(r, 17)
            acc[i] = jnp.maximum(acc[i], qbuf[j])
            return 0

        lax.fori_loop(0, kc, rmw, 0)
        return 0

    lax.fori_loop(0, nch, chunk, 0)

    pltpu.sync_copy(acc, s_hbm.at[pl.ds(base, ROWS)])


def _segmax(binned, deg, qp):
    mesh = plsc.VectorSubcoreMesh(core_axis_name="c", subcore_axis_name="s")
    return pl.kernel(
        _segmax_body,
        out_type=jax.ShapeDtypeStruct((NPAD, DP), jnp.float32),
        mesh=mesh,
        compiler_params=_SC_PARAMS,
        scratch_types=[
            pltpu.VMEM((GCHUNK + 16,), jnp.int32),
            pltpu.VMEM((GCHUNK,), jnp.int32),
            pltpu.VMEM((GCHUNK, DP), jnp.float32),
            pltpu.VMEM((ROWS, DP), jnp.float32),
            pltpu.VMEM((48,), jnp.int32),
            pltpu.SemaphoreType.DMA,
        ],
    )(binned, deg, qp)


def _pre_body(x_ref, w1_ref, b1_ref, qp_ref, p2_ref):
    x = x_ref[...]
    w1 = w1_ref[...]
    a = w1[:, :D]
    b = w1[:, D:]
    q = jnp.dot(x, b.T, preferred_element_type=jnp.float32)
    qp_ref[...] = jnp.pad(q, ((0, 0), (0, DP - DOUT)))
    p2_ref[...] = jnp.dot(x, (a - b).T, preferred_element_type=jnp.float32) + b1_ref[...]


def _pre(x, W1, b1):
    BN = 10000
    return pl.pallas_call(
        _pre_body,
        grid=(N // BN,),
        in_specs=[
            pl.BlockSpec((BN, D), lambda i: (i, 0)),
            pl.BlockSpec((DOUT, 2 * D), lambda i: (0, 0)),
            pl.BlockSpec((DOUT,), lambda i: (0,)),
        ],
        out_specs=[
            pl.BlockSpec((BN, DP), lambda i: (i, 0)),
            pl.BlockSpec((BN, DOUT), lambda i: (i, 0)),
        ],
        out_shape=[
            jax.ShapeDtypeStruct((NPAD, DP), jnp.float32),
            jax.ShapeDtypeStruct((N, DOUT), jnp.float32),
        ],
    )(x, W1, b1)


def _post_body(s_ref, p2_ref, w2_ref, b2_ref, out_ref):
    s = s_ref[...][:, :DOUT]
    agg = jnp.where(jnp.isneginf(s), 0.0, p2_ref[...] + s)
    out_ref[...] = jnp.dot(agg, w2_ref[...].T, preferred_element_type=jnp.float32) + b2_ref[...]


def _post(s, p2, W2, b2):
    BN = 10000
    return pl.pallas_call(
        _post_body,
        grid=(N // BN,),
        in_specs=[
            pl.BlockSpec((BN, DP), lambda i: (i, 0)),
            pl.BlockSpec((BN, DOUT), lambda i: (i, 0)),
            pl.BlockSpec((DOUT, DOUT), lambda i: (0, 0)),
            pl.BlockSpec((DOUT,), lambda i: (0,)),
        ],
        out_specs=pl.BlockSpec((BN, DOUT), lambda i: (i, 0)),
        out_shape=jax.ShapeDtypeStruct((N, DOUT), jnp.float32),
    )(s, p2, W2, b2)


def kernel(x, edge_index, mask, W1, b1, W2, b2):
    src = edge_index[0]
    dst = edge_index[1]
    dst3 = dst.reshape(NECH, 1, ECHUNK)
    src3 = src.reshape(NECH, 1, ECHUNK)
    rec3, sidx3, deg2 = _prep(dst3, src3)
    rec = rec3.reshape(E)
    sidx = sidx3.reshape(E)
    deg = jnp.pad(deg2[:, 0], (0, 16))
    binned = _scatter(rec, sidx)
    qp, p2 = _pre(x, W1, b1)
    s = _segmax(binned, deg, qp)
    return _post(s, p2, W2, b2)
